# baseline (speedup 1.0000x reference)
import jax
import jax.numpy as jnp
from jax import lax
from jax.experimental import pallas as pl
from jax.experimental.pallas import tpu as pltpu
from jax.experimental.pallas import tpu_sc as plsc

B = 4
T = 2048
E = 256
P = 16
L = T // P
W = 32
JT = W + L
M = P * B
ROWS = JT * M
NW = 32
GW = ROWS // NW
GISS = (64, 64, 64, 64, 64)
JCH = 32
NJCH = JT // JCH
GXCH = 1280
MIXN = (B * T) // NW


def _sc_mesh():
    return plsc.VectorSubcoreMesh(core_axis_name="c", subcore_axis_name="s")


def _gather_body(tbl, idxh, out, idx_v, rows_v, sg, so):
    cid = lax.axis_index("c")
    sid = lax.axis_index("s")
    w = sid * 2 + cid
    base = w * GW
    offs = []
    off = 0
    for k, sz in enumerate(GISS):
        pltpu.sync_copy(idxh.at[pl.ds(base + off, sz)], idx_v[k])
        offs.append(off)
        off += sz
    gs = [pltpu.async_copy(tbl.at[idx_v[k]], rows_v[k], sg[k])
          for k in range(len(GISS))]
    os_ = []
    for k, sz in enumerate(GISS):
        gs[k].wait()
        os_.append(pltpu.async_copy(
            rows_v[k], out.at[pl.ds(base + offs[k], sz)], so[k]))
    for o in os_:
        o.wait()


def _word_gather(word_table, idx_flat):
    f = pl.kernel(
        _gather_body,
        out_type=jax.ShapeDtypeStruct((ROWS, E), jnp.float32),
        mesh=_sc_mesh(),
        scratch_types=[
            [pltpu.VMEM((sz,), jnp.int32) for sz in GISS],
            [pltpu.VMEM((sz, E), jnp.float32) for sz in GISS],
            [pltpu.SemaphoreType.DMA for _ in GISS],
            [pltpu.SemaphoreType.DMA for _ in GISS],
        ],
    )
    return f(word_table, idx_flat)


def _gx_body(tag, posq, ptab, wxp, wxw, bias, out, pg):
    @pl.when(pl.program_id(0) == 0)
    def _():
        pg[...] = jnp.dot(ptab[...], wxp[...],
                          preferred_element_type=jnp.float32
                          ).astype(jnp.bfloat16)

    rows = tag.shape[0]
    oh = (posq[...] == lax.broadcasted_iota(jnp.int32, (rows, 64), 1)).astype(
        jnp.bfloat16)
    out[...] = (jnp.dot(tag[...].astype(jnp.bfloat16), wxw[...],
                        preferred_element_type=jnp.float32)
                + jnp.dot(oh, pg[...],
                          preferred_element_type=jnp.float32)
                + bias[...])


def _gx(tag, posq, pos_table, wxp, wxw, bias):
    return pl.pallas_call(
        _gx_body,
        grid=(ROWS // GXCH,),
        in_specs=[
            pl.BlockSpec((GXCH, E), lambda i: (i, 0)),
            pl.BlockSpec((GXCH, 1), lambda i: (i, 0)),
            pl.BlockSpec((64, E), lambda i: (0, 0)),
            pl.BlockSpec((E, 3 * E), lambda i: (0, 0)),
            pl.BlockSpec((E, 3 * E), lambda i: (0, 0)),
            pl.BlockSpec((1, 3 * E), lambda i: (0, 0)),
        ],
        out_specs=pl.BlockSpec((GXCH, 3 * E), lambda i: (i, 0)),
        name="gx_matmul",
        out_shape=jax.ShapeDtypeStruct((ROWS, 3 * E), jnp.float32),
        scratch_shapes=[pltpu.VMEM((64, 3 * E), jnp.bfloat16)],
    )(tag, posq, pos_table, wxp, wxw, bias)


def _gru_body(gx, wh, bhn, wo, s_out, h_ref, hs_ref):
    @pl.when(pl.program_id(0) == 0)
    def _():
        h_ref[...] = jnp.zeros((M, E), jnp.float32)

    @pl.when(pl.program_id(0) == W // JCH)
    def _():
        h_ref[0:B, :] = jnp.zeros((B, E), jnp.float32)

    bhnv = bhn[...]

    def step(i, h):
        g = gx[pl.ds(i * M, M), :]
        gh = jnp.dot(h.astype(jnp.bfloat16), wh[...],
                     preferred_element_type=jnp.float32)
        r = jax.nn.sigmoid(g[:, 0:E] + gh[:, 0:E])
        z = jax.nn.sigmoid(g[:, E:2 * E] + gh[:, E:2 * E])
        n = jnp.tanh(g[:, 2 * E:3 * E] + r * (gh[:, 2 * E:3 * E] + bhnv))
        h2 = (1.0 - z) * n + z * h
        hs_ref[pl.ds(i * M, M), :] = h2
        return h2

    h = lax.fori_loop(0, JCH, step, h_ref[...], unroll=4)
    h_ref[...] = h
    s_out[...] = jnp.sum(hs_ref[...] * wo[...], axis=1, keepdims=True)


def _gru(gxa, wh, bhn, wo_row):
    rows = JCH * M
    return pl.pallas_call(
        _gru_body,
        grid=(NJCH,),
        in_specs=[
            pl.BlockSpec((rows, 3 * E), lambda i: (i, 0)),
            pl.BlockSpec((E, 3 * E), lambda i: (0, 0)),
            pl.BlockSpec((1, E), lambda i: (0, 0)),
            pl.BlockSpec((1, E), lambda i: (0, 0)),
        ],
        out_specs=pl.BlockSpec((rows, 1), lambda i: (i, 0)),
        out_shape=jax.ShapeDtypeStruct((ROWS, 1), jnp.float32),
        scratch_shapes=[
            pltpu.VMEM((M, E), jnp.float32),
            pltpu.VMEM((rows, E), jnp.float32),
        ],
    )(gxa, wh, bhn, wo_row)


def _mix_body(s_h, sidx_h, aidx_h, out_h, s_v, sidx_v, aidx_v, out_v):
    cid = lax.axis_index("c")
    sid = lax.axis_index("s")
    w = sid * 2 + cid
    pltpu.sync_copy(s_h, s_v)
    pltpu.sync_copy(sidx_h.at[w], sidx_v)
    pltpu.sync_copy(aidx_h.at[w], aidx_v)
    for j in range(MIXN // 16):
        si = sidx_v[pl.ds(j * 16, 16)]
        ai = aidx_v[pl.ds(j * 16, 16)]
        gs = plsc.load_gather(s_v, [si])
        ga = plsc.load_gather(s_v, [ai])
        out_v[pl.ds(j * 16, 16)] = 0.5 * (gs + ga)
    pltpu.sync_copy(out_v, out_h.at[pl.ds(w * MIXN, MIXN)])


def _mix(s_flat, self_rs, ante_rs):
    f = pl.kernel(
        _mix_body,
        out_type=jax.ShapeDtypeStruct((B * T,), jnp.float32),
        mesh=_sc_mesh(),
        compiler_params=pltpu.CompilerParams(needs_layout_passes=False),
        scratch_types=[
            pltpu.VMEM((ROWS,), jnp.float32),
            pltpu.VMEM((MIXN,), jnp.int32),
            pltpu.VMEM((MIXN,), jnp.int32),
            pltpu.VMEM((MIXN,), jnp.float32),
        ],
    )
    return f(s_flat, self_rs, ante_rs)


def _sm_body(m, o):
    v = m[...]
    mx = jnp.max(v, axis=1, keepdims=True)
    e = jnp.exp(v - mx)
    o[...] = e / jnp.sum(e, axis=1, keepdims=True)


def _softmax(mix2):
    return pl.pallas_call(
        _sm_body,
        out_shape=jax.ShapeDtypeStruct((B, T), jnp.float32),
    )(mix2)


def kernel(x, co, pos_table, word_table, Wx, Wh, bx, bh, Wo, bo):
    pos_idx = x[:, :, 0]
    word_idx = x[:, :, 1]

    def seg_layout(idx_bt):
        it = idx_bt.T
        warm = jnp.roll(it, W, axis=0).reshape(P, L, B)[:, :W]
        main = it.reshape(P, L, B)
        return jnp.concatenate([warm, main], axis=1).transpose(1, 0, 2)

    widx2 = seg_layout(word_idx)
    pidx2 = seg_layout(pos_idx)

    tag = _word_gather(word_table, widx2.reshape(ROWS))

    bias = (bx + jnp.concatenate([bh[:2 * E], jnp.zeros((E,), jnp.float32)]))
    gxa = _gx(tag, pidx2.reshape(ROWS, 1), pos_table, Wx[:E],
              Wx[E:].astype(jnp.bfloat16), bias.reshape(1, 3 * E))

    s = _gru(gxa, Wh.astype(jnp.bfloat16), bh[2 * E:].reshape(1, E),
             Wo.reshape(1, E))

    tt = jnp.arange(T)
    row_t = ((W + tt % L) * P + tt // L) * B
    self_idx = (row_t[None, :] + jnp.arange(B)[:, None])
    ante_idx = ((W + co % L) * P + co // L) * B + jnp.arange(B)[:, None]
    mix = _mix(s.reshape(ROWS), self_idx.reshape(NW, MIXN),
               ante_idx.reshape(NW, MIXN))

    return _softmax(mix.reshape(B, T))


# speedup vs baseline: 1.0536x; 1.0536x over previous
import jax
import jax.numpy as jnp
from jax import lax
from jax.experimental import pallas as pl
from jax.experimental.pallas import tpu as pltpu
from jax.experimental.pallas import tpu_sc as plsc

B = 4
T = 2048
E = 256
P = 16
L = T // P
W = 32
JT = W + L
M = P * B
ROWS = JT * M
NW = 32
GW = ROWS // NW
GISS = (128, 128, 64)
JCH = 32
NJCH = JT // JCH
GXCH = 1280
MIXN = (B * T) // NW


def _sc_mesh():
    return plsc.VectorSubcoreMesh(core_axis_name="c", subcore_axis_name="s")


def _gather_body(tbl, idxh, out, idx_v, rows_v, sg, so):
    cid = lax.axis_index("c")
    sid = lax.axis_index("s")
    w = sid * 2 + cid
    base = w * GW
    offs = []
    off = 0
    for k, sz in enumerate(GISS):
        pltpu.sync_copy(idxh.at[pl.ds(base + off, sz)], idx_v[k])
        offs.append(off)
        off += sz
    gs = [pltpu.async_copy(tbl.at[idx_v[k]], rows_v[k], sg[k])
          for k in range(len(GISS))]
    os_ = []
    for k, sz in enumerate(GISS):
        gs[k].wait()
        os_.append(pltpu.async_copy(
            rows_v[k], out.at[pl.ds(base + offs[k], sz)], so[k]))
    for o in os_:
        o.wait()


def _word_gather(word_table, idx_flat):
    f = pl.kernel(
        _gather_body,
        out_type=jax.ShapeDtypeStruct((ROWS, E), jnp.float32),
        mesh=_sc_mesh(),
        scratch_types=[
            [pltpu.VMEM((sz,), jnp.int32) for sz in GISS],
            [pltpu.VMEM((sz, E), jnp.float32) for sz in GISS],
            [pltpu.SemaphoreType.DMA for _ in GISS],
            [pltpu.SemaphoreType.DMA for _ in GISS],
        ],
    )
    return f(word_table, idx_flat)


def _gx_body(tag, posq, ptab, wxp, wxw, bias, out, pg):
    @pl.when(pl.program_id(0) == 0)
    def _():
        pg[...] = jnp.dot(ptab[...], wxp[...],
                          preferred_element_type=jnp.float32
                          ).astype(jnp.bfloat16)

    rows = tag.shape[0]
    oh = (posq[...] == lax.broadcasted_iota(jnp.int32, (rows, 64), 1)).astype(
        jnp.bfloat16)
    out[...] = (jnp.dot(tag[...].astype(jnp.bfloat16), wxw[...],
                        preferred_element_type=jnp.float32)
                + jnp.dot(oh, pg[...],
                          preferred_element_type=jnp.float32)
                + bias[...]).astype(jnp.bfloat16)


def _gx(tag, posq, pos_table, wxp, wxw, bias):
    return pl.pallas_call(
        _gx_body,
        grid=(ROWS // GXCH,),
        in_specs=[
            pl.BlockSpec((GXCH, E), lambda i: (i, 0)),
            pl.BlockSpec((GXCH, 1), lambda i: (i, 0)),
            pl.BlockSpec((64, E), lambda i: (0, 0)),
            pl.BlockSpec((E, 3 * E), lambda i: (0, 0)),
            pl.BlockSpec((E, 3 * E), lambda i: (0, 0)),
            pl.BlockSpec((1, 3 * E), lambda i: (0, 0)),
        ],
        out_specs=pl.BlockSpec((GXCH, 3 * E), lambda i: (i, 0)),
        name="gx_matmul",
        out_shape=jax.ShapeDtypeStruct((ROWS, 3 * E), jnp.bfloat16),
        scratch_shapes=[pltpu.VMEM((64, 3 * E), jnp.bfloat16)],
    )(tag, posq, pos_table, wxp, wxw, bias)


def _gru_body(gx, wh, bhn, wo, s_out, h_ref, hs_ref):
    @pl.when(pl.program_id(0) == 0)
    def _():
        h_ref[...] = jnp.zeros((M, E), jnp.float32)

    @pl.when(pl.program_id(0) == W // JCH)
    def _():
        h_ref[0:B, :] = jnp.zeros((B, E), jnp.float32)

    bhnv = bhn[...]
    wov = wo[...]

    def step(i, h):
        g = gx[pl.ds(i * M, M), :]
        gh = jnp.dot(h.astype(jnp.bfloat16), wh[...],
                     preferred_element_type=jnp.float32)
        r = 0.5 + 0.5 * jnp.tanh(0.5 * (g[:, 0:E] + gh[:, 0:E]))
        z = 0.5 + 0.5 * jnp.tanh(0.5 * (g[:, E:2 * E] + gh[:, E:2 * E]))
        n = jnp.tanh(g[:, 2 * E:3 * E] + r * (gh[:, 2 * E:3 * E] + bhnv))
        h2 = n + z * (h - n)
        hs_ref[pl.ds(i * M, M), :] = h2
        return h2

    h_ref[...] = lax.fori_loop(0, JCH, step, h_ref[...], unroll=8)
    s_out[...] = jnp.sum(hs_ref[...] * wov, axis=1, keepdims=True)


def _gru(gxa, wh, bhn, wo_row):
    rows = JCH * M
    return pl.pallas_call(
        _gru_body,
        grid=(NJCH,),
        in_specs=[
            pl.BlockSpec((rows, 3 * E), lambda i: (i, 0)),
            pl.BlockSpec((E, 3 * E), lambda i: (0, 0)),
            pl.BlockSpec((1, E), lambda i: (0, 0)),
            pl.BlockSpec((1, E), lambda i: (0, 0)),
        ],
        out_specs=pl.BlockSpec((rows, 1), lambda i: (i, 0)),
        out_shape=jax.ShapeDtypeStruct((ROWS, 1), jnp.float32),
        scratch_shapes=[
            pltpu.VMEM((M, E), jnp.float32),
            pltpu.VMEM((rows, E), jnp.float32),
        ],
    )(gxa, wh, bhn, wo_row)


def _mix_body(s_h, sidx_h, aidx_h, out_h, s_v, sidx_v, aidx_v, out_v):
    cid = lax.axis_index("c")
    sid = lax.axis_index("s")
    w = sid * 2 + cid
    pltpu.sync_copy(s_h, s_v)
    pltpu.sync_copy(sidx_h.at[w], sidx_v)
    pltpu.sync_copy(aidx_h.at[w], aidx_v)
    for j in range(MIXN // 16):
        si = sidx_v[pl.ds(j * 16, 16)]
        ai = aidx_v[pl.ds(j * 16, 16)]
        gs = plsc.load_gather(s_v, [si])
        ga = plsc.load_gather(s_v, [ai])
        out_v[pl.ds(j * 16, 16)] = 0.5 * (gs + ga)
    pltpu.sync_copy(out_v, out_h.at[pl.ds(w * MIXN, MIXN)])


def _mix(s_flat, self_rs, ante_rs):
    f = pl.kernel(
        _mix_body,
        out_type=jax.ShapeDtypeStruct((B * T,), jnp.float32),
        mesh=_sc_mesh(),
        compiler_params=pltpu.CompilerParams(needs_layout_passes=False),
        scratch_types=[
            pltpu.VMEM((ROWS,), jnp.float32),
            pltpu.VMEM((MIXN,), jnp.int32),
            pltpu.VMEM((MIXN,), jnp.int32),
            pltpu.VMEM((MIXN,), jnp.float32),
        ],
    )
    return f(s_flat, self_rs, ante_rs)


def _sm_body(m, o):
    v = m[...]
    mx = jnp.max(v, axis=1, keepdims=True)
    e = jnp.exp(v - mx)
    o[...] = e / jnp.sum(e, axis=1, keepdims=True)


def _softmax(mix2):
    return pl.pallas_call(
        _sm_body,
        out_shape=jax.ShapeDtypeStruct((B, T), jnp.float32),
    )(mix2)


def kernel(x, co, pos_table, word_table, Wx, Wh, bx, bh, Wo, bo):
    pos_idx = x[:, :, 0]
    word_idx = x[:, :, 1]

    def seg_layout(idx_bt):
        it = idx_bt.T
        warm = jnp.roll(it, W, axis=0).reshape(P, L, B)[:, :W]
        main = it.reshape(P, L, B)
        return jnp.concatenate([warm, main], axis=1).transpose(1, 0, 2)

    widx2 = seg_layout(word_idx)
    pidx2 = seg_layout(pos_idx)

    tag = _word_gather(word_table, widx2.reshape(ROWS))

    bias = (bx + jnp.concatenate([bh[:2 * E], jnp.zeros((E,), jnp.float32)]))
    gxa = _gx(tag, pidx2.reshape(ROWS, 1), pos_table, Wx[:E],
              Wx[E:].astype(jnp.bfloat16), bias.reshape(1, 3 * E))

    s = _gru(gxa, Wh.astype(jnp.bfloat16), bh[2 * E:].reshape(1, E),
             Wo.reshape(1, E))

    tt = jnp.arange(T)
    row_t = ((W + tt % L) * P + tt // L) * B
    self_idx = (row_t[None, :] + jnp.arange(B)[:, None])
    ante_idx = ((W + co % L) * P + co // L) * B + jnp.arange(B)[:, None]
    mix = _mix(s.reshape(ROWS), self_idx.reshape(NW, MIXN),
               ante_idx.reshape(NW, MIXN))

    return _softmax(mix.reshape(B, T))


# speedup vs baseline: 1.1263x; 1.0690x over previous
import jax
import jax.numpy as jnp
from jax import lax
from jax.experimental import pallas as pl
from jax.experimental.pallas import tpu as pltpu
from jax.experimental.pallas import tpu_sc as plsc

B = 4
T = 2048
E = 256
P = 16
L = T // P
W = 32
JT = W + L
M = P * B
ROWS = JT * M
NW = 32
GW = ROWS // NW
GISS = (128, 128, 64)
JCH = 32
NJCH = JT // JCH
GXCH = 1280
MIXN = (B * T) // NW


def _sc_mesh():
    return plsc.VectorSubcoreMesh(core_axis_name="c", subcore_axis_name="s")


def _gather_body(tbl, idxh, out, idx_v, rows_v, sg, so):
    cid = lax.axis_index("c")
    sid = lax.axis_index("s")
    w = sid * 2 + cid
    base = w * GW
    offs = []
    off = 0
    for k, sz in enumerate(GISS):
        pltpu.sync_copy(idxh.at[pl.ds(base + off, sz)], idx_v[k])
        offs.append(off)
        off += sz
    gs = [pltpu.async_copy(tbl.at[idx_v[k]], rows_v[k], sg[k])
          for k in range(len(GISS))]
    os_ = []
    for k, sz in enumerate(GISS):
        gs[k].wait()
        os_.append(pltpu.async_copy(
            rows_v[k], out.at[pl.ds(base + offs[k], sz)], so[k]))
    for o in os_:
        o.wait()


def _word_gather(word_table, idx_flat):
    f = pl.kernel(
        _gather_body,
        out_type=jax.ShapeDtypeStruct((ROWS, E), jnp.float32),
        mesh=_sc_mesh(),
        scratch_types=[
            [pltpu.VMEM((sz,), jnp.int32) for sz in GISS],
            [pltpu.VMEM((sz, E), jnp.float32) for sz in GISS],
            [pltpu.SemaphoreType.DMA for _ in GISS],
            [pltpu.SemaphoreType.DMA for _ in GISS],
        ],
    )
    return f(word_table, idx_flat)


def _gru_body(tag, posq, ptab, wxp, wxw, bias, wh, bhn, wo, s_out,
              h_ref, hs_ref, pg, gx_s):
    @pl.when(pl.program_id(0) == 0)
    def _():
        h_ref[...] = jnp.zeros((M, E), jnp.float32)
        pg[...] = jnp.dot(ptab[...], wxp[...],
                          preferred_element_type=jnp.float32
                          ).astype(jnp.bfloat16)

    @pl.when(pl.program_id(0) == W // JCH)
    def _():
        h_ref[0:B, :] = jnp.zeros((B, E), jnp.float32)

    rows = tag.shape[0]
    oh = (posq[...] == lax.broadcasted_iota(jnp.int32, (rows, 64), 1)).astype(
        jnp.bfloat16)
    gx_s[...] = (jnp.dot(tag[...].astype(jnp.bfloat16), wxw[...],
                         preferred_element_type=jnp.float32)
                 + jnp.dot(oh, pg[...],
                           preferred_element_type=jnp.float32)
                 + bias[...]).astype(jnp.bfloat16)

    bhnv = bhn[...]
    wov = wo[...]

    def step(i, h):
        g = gx_s[pl.ds(i * M, M), :]
        gh = jnp.dot(h.astype(jnp.bfloat16), wh[...],
                     preferred_element_type=jnp.float32)
        r = 0.5 + 0.5 * jnp.tanh(0.5 * (g[:, 0:E] + gh[:, 0:E]))
        z = 0.5 + 0.5 * jnp.tanh(0.5 * (g[:, E:2 * E] + gh[:, E:2 * E]))
        n = jnp.tanh(g[:, 2 * E:3 * E] + r * (gh[:, 2 * E:3 * E] + bhnv))
        h2 = n + z * (h - n)
        hs_ref[pl.ds(i * M, M), :] = h2
        return h2

    h_ref[...] = lax.fori_loop(0, JCH, step, h_ref[...], unroll=8)
    s_out[...] = jnp.sum(hs_ref[...] * wov, axis=1, keepdims=True)


def _gru(tag, posq, pos_table, wxp, wxw, bias, wh, bhn, wo_row):
    rows = JCH * M
    return pl.pallas_call(
        _gru_body,
        grid=(NJCH,),
        in_specs=[
            pl.BlockSpec((rows, E), lambda i: (i, 0)),
            pl.BlockSpec((rows, 1), lambda i: (i, 0)),
            pl.BlockSpec((64, E), lambda i: (0, 0)),
            pl.BlockSpec((E, 3 * E), lambda i: (0, 0)),
            pl.BlockSpec((E, 3 * E), lambda i: (0, 0)),
            pl.BlockSpec((1, 3 * E), lambda i: (0, 0)),
            pl.BlockSpec((E, 3 * E), lambda i: (0, 0)),
            pl.BlockSpec((1, E), lambda i: (0, 0)),
            pl.BlockSpec((1, E), lambda i: (0, 0)),
        ],
        out_specs=pl.BlockSpec((rows, 1), lambda i: (i, 0)),
        out_shape=jax.ShapeDtypeStruct((ROWS, 1), jnp.float32),
        scratch_shapes=[
            pltpu.VMEM((M, E), jnp.float32),
            pltpu.VMEM((rows, E), jnp.float32),
            pltpu.VMEM((64, 3 * E), jnp.bfloat16),
            pltpu.VMEM((rows, 3 * E), jnp.bfloat16),
        ],
    )(tag, posq, pos_table, wxp, wxw, bias, wh, bhn, wo_row)


def _mix_body(s_h, sidx_h, aidx_h, out_h, s_v, sidx_v, aidx_v, out_v):
    cid = lax.axis_index("c")
    sid = lax.axis_index("s")
    w = sid * 2 + cid
    pltpu.sync_copy(s_h, s_v)
    pltpu.sync_copy(sidx_h.at[w], sidx_v)
    pltpu.sync_copy(aidx_h.at[w], aidx_v)
    for j in range(MIXN // 16):
        si = sidx_v[pl.ds(j * 16, 16)]
        ai = aidx_v[pl.ds(j * 16, 16)]
        gs = plsc.load_gather(s_v, [si])
        ga = plsc.load_gather(s_v, [ai])
        out_v[pl.ds(j * 16, 16)] = 0.5 * (gs + ga)
    pltpu.sync_copy(out_v, out_h.at[pl.ds(w * MIXN, MIXN)])


def _mix(s_flat, self_rs, ante_rs):
    f = pl.kernel(
        _mix_body,
        out_type=jax.ShapeDtypeStruct((B * T,), jnp.float32),
        mesh=_sc_mesh(),
        compiler_params=pltpu.CompilerParams(needs_layout_passes=False),
        scratch_types=[
            pltpu.VMEM((ROWS,), jnp.float32),
            pltpu.VMEM((MIXN,), jnp.int32),
            pltpu.VMEM((MIXN,), jnp.int32),
            pltpu.VMEM((MIXN,), jnp.float32),
        ],
    )
    return f(s_flat, self_rs, ante_rs)


def _sm_body(m, o):
    v = m[...]
    mx = jnp.max(v, axis=1, keepdims=True)
    e = jnp.exp(v - mx)
    o[...] = e / jnp.sum(e, axis=1, keepdims=True)


def _softmax(mix2):
    return pl.pallas_call(
        _sm_body,
        out_shape=jax.ShapeDtypeStruct((B, T), jnp.float32),
    )(mix2)


def kernel(x, co, pos_table, word_table, Wx, Wh, bx, bh, Wo, bo):
    pos_idx = x[:, :, 0]
    word_idx = x[:, :, 1]

    def seg_layout(idx_bt):
        it = idx_bt.T
        warm = jnp.roll(it, W, axis=0).reshape(P, L, B)[:, :W]
        main = it.reshape(P, L, B)
        return jnp.concatenate([warm, main], axis=1).transpose(1, 0, 2)

    widx2 = seg_layout(word_idx)
    pidx2 = seg_layout(pos_idx)

    tag = _word_gather(word_table, widx2.reshape(ROWS))

    bias = (bx + jnp.concatenate([bh[:2 * E], jnp.zeros((E,), jnp.float32)]))
    s = _gru(tag, pidx2.reshape(ROWS, 1), pos_table, Wx[:E],
             Wx[E:].astype(jnp.bfloat16), bias.reshape(1, 3 * E),
             Wh.astype(jnp.bfloat16), bh[2 * E:].reshape(1, E),
             Wo.reshape(1, E))

    tt = jnp.arange(T)
    row_t = ((W + tt % L) * P + tt // L) * B
    self_idx = (row_t[None, :] + jnp.arange(B)[:, None])
    ante_idx = ((W + co % L) * P + co // L) * B + jnp.arange(B)[:, None]
    mix = _mix(s.reshape(ROWS), self_idx.reshape(NW, MIXN),
               ante_idx.reshape(NW, MIXN))

    return _softmax(mix.reshape(B, T))


# speedup vs baseline: 1.1400x; 1.0122x over previous
import jax
import jax.numpy as jnp
from jax import lax
from jax.experimental import pallas as pl
from jax.experimental.pallas import tpu as pltpu
from jax.experimental.pallas import tpu_sc as plsc

B = 4
T = 2048
E = 256
P = 16
L = T // P
W = 32
JT = W + L
M = P * B
ROWS = JT * M
NW = 32
GW = ROWS // NW
GISS = (128, 128, 64)
JCH = 32
NJCH = JT // JCH
GXCH = 1280
MIXN = (B * T) // NW


def _sc_mesh():
    return plsc.VectorSubcoreMesh(core_axis_name="c", subcore_axis_name="s")


def _gather_body(tbl, idxh, out, idx_v, rows_v, sg, so):
    cid = lax.axis_index("c")
    sid = lax.axis_index("s")
    w = sid * 2 + cid
    base = w * GW
    offs = []
    off = 0
    for k, sz in enumerate(GISS):
        pltpu.sync_copy(idxh.at[pl.ds(base + off, sz)], idx_v[k])
        offs.append(off)
        off += sz
    gs = [pltpu.async_copy(tbl.at[idx_v[k]], rows_v[k], sg[k])
          for k in range(len(GISS))]
    os_ = []
    for k, sz in enumerate(GISS):
        gs[k].wait()
        os_.append(pltpu.async_copy(
            rows_v[k], out.at[pl.ds(base + offs[k], sz)], so[k]))
    for o in os_:
        o.wait()


def _word_gather(word_table, idx_flat):
    f = pl.kernel(
        _gather_body,
        out_type=jax.ShapeDtypeStruct((ROWS, E), jnp.float32),
        mesh=_sc_mesh(),
        scratch_types=[
            [pltpu.VMEM((sz,), jnp.int32) for sz in GISS],
            [pltpu.VMEM((sz, E), jnp.float32) for sz in GISS],
            [pltpu.SemaphoreType.DMA for _ in GISS],
            [pltpu.SemaphoreType.DMA for _ in GISS],
        ],
    )
    return f(word_table, idx_flat)


def _gru_body(tag, posq, ptab, wxp, wxw, bias, wh, bhn, wo, s_out,
              h_ref, hs_ref, pg, gx_s):
    @pl.when(pl.program_id(0) == 0)
    def _():
        h_ref[...] = jnp.zeros((M, E), jnp.float32)
        pg[...] = jnp.dot(ptab[...], wxp[...],
                          preferred_element_type=jnp.float32
                          ).astype(jnp.bfloat16)

    @pl.when(pl.program_id(0) == W // JCH)
    def _():
        h_ref[0:B, :] = jnp.zeros((B, E), jnp.float32)

    rows = tag.shape[0]
    oh = (posq[...] == lax.broadcasted_iota(jnp.int32, (rows, 64), 1)).astype(
        jnp.bfloat16)
    gx_s[...] = (jnp.dot(tag[...].astype(jnp.bfloat16), wxw[...],
                         preferred_element_type=jnp.float32)
                 + jnp.dot(oh, pg[...],
                           preferred_element_type=jnp.float32)
                 + bias[...]).astype(jnp.bfloat16)

    bhnv = bhn[...]
    wov = wo[...]

    def step(i, h):
        g = gx_s[pl.ds(i * M, M), :]
        gh = jnp.dot(h.astype(jnp.bfloat16), wh[...],
                     preferred_element_type=jnp.float32)
        r = 0.5 + 0.5 * jnp.tanh(0.5 * (g[:, 0:E] + gh[:, 0:E]))
        z = 0.5 + 0.5 * jnp.tanh(0.5 * (g[:, E:2 * E] + gh[:, E:2 * E]))
        n = jnp.tanh(g[:, 2 * E:3 * E] + r * (gh[:, 2 * E:3 * E] + bhnv))
        h2 = n + z * (h - n)
        hs_ref[pl.ds(i * M, M), :] = h2
        return h2

    h_ref[...] = lax.fori_loop(0, JCH, step, h_ref[...], unroll=8)
    s_out[...] = jnp.sum(hs_ref[...] * wov, axis=1, keepdims=True)


def _gru(tag, posq, pos_table, wxp, wxw, bias, wh, bhn, wo_row):
    rows = JCH * M
    return pl.pallas_call(
        _gru_body,
        grid=(NJCH,),
        in_specs=[
            pl.BlockSpec((rows, E), lambda i: (i, 0)),
            pl.BlockSpec((rows, 1), lambda i: (i, 0)),
            pl.BlockSpec((64, E), lambda i: (0, 0)),
            pl.BlockSpec((E, 3 * E), lambda i: (0, 0)),
            pl.BlockSpec((E, 3 * E), lambda i: (0, 0)),
            pl.BlockSpec((1, 3 * E), lambda i: (0, 0)),
            pl.BlockSpec((E, 3 * E), lambda i: (0, 0)),
            pl.BlockSpec((1, E), lambda i: (0, 0)),
            pl.BlockSpec((1, E), lambda i: (0, 0)),
        ],
        out_specs=pl.BlockSpec((rows, 1), lambda i: (i, 0)),
        out_shape=jax.ShapeDtypeStruct((ROWS, 1), jnp.float32),
        scratch_shapes=[
            pltpu.VMEM((M, E), jnp.float32),
            pltpu.VMEM((rows, E), jnp.float32),
            pltpu.VMEM((64, 3 * E), jnp.bfloat16),
            pltpu.VMEM((rows, 3 * E), jnp.bfloat16),
        ],
    )(tag, posq, pos_table, wxp, wxw, bias, wh, bhn, wo_row)


def _mix_body(s_h, sidx_h, aidx_h, out_h, s_v, sidx_v, aidx_v, out_v):
    cid = lax.axis_index("c")
    sid = lax.axis_index("s")
    w = sid * 2 + cid
    pltpu.sync_copy(s_h, s_v)
    pltpu.sync_copy(sidx_h.at[w], sidx_v)
    pltpu.sync_copy(aidx_h.at[w], aidx_v)
    for j in range(MIXN // 16):
        si = sidx_v[pl.ds(j * 16, 16)]
        ai = aidx_v[pl.ds(j * 16, 16)]
        gs = plsc.load_gather(s_v, [si])
        ga = plsc.load_gather(s_v, [ai])
        out_v[pl.ds(j * 16, 16)] = 0.5 * (gs + ga)
    b = w // (T // MIXN)
    tcol = (w % (T // MIXN)) * MIXN
    pltpu.sync_copy(out_v, out_h.at[b, pl.ds(tcol, MIXN)])


def _mix(s_flat, self_rs, ante_rs):
    f = pl.kernel(
        _mix_body,
        out_type=jax.ShapeDtypeStruct((B, T), jnp.float32),
        mesh=_sc_mesh(),
        compiler_params=pltpu.CompilerParams(needs_layout_passes=False),
        scratch_types=[
            pltpu.VMEM((ROWS,), jnp.float32),
            pltpu.VMEM((MIXN,), jnp.int32),
            pltpu.VMEM((MIXN,), jnp.int32),
            pltpu.VMEM((MIXN,), jnp.float32),
        ],
    )
    return f(s_flat, self_rs, ante_rs)


def _sm_body(m, o):
    v = m[...]
    mx = jnp.max(v, axis=1, keepdims=True)
    e = jnp.exp(v - mx)
    o[...] = e / jnp.sum(e, axis=1, keepdims=True)


def _softmax(mix2):
    return pl.pallas_call(
        _sm_body,
        out_shape=jax.ShapeDtypeStruct((B, T), jnp.float32),
    )(mix2)


def kernel(x, co, pos_table, word_table, Wx, Wh, bx, bh, Wo, bo):
    pos_idx = x[:, :, 0]
    word_idx = x[:, :, 1]

    def seg_layout(idx_bt):
        it = idx_bt.T
        warm = jnp.roll(it, W, axis=0).reshape(P, L, B)[:, :W]
        main = it.reshape(P, L, B)
        return jnp.concatenate([warm, main], axis=1).transpose(1, 0, 2)

    widx2 = seg_layout(word_idx)
    pidx2 = seg_layout(pos_idx)

    tag = _word_gather(word_table, widx2.reshape(ROWS))

    bias = (bx + jnp.concatenate([bh[:2 * E], jnp.zeros((E,), jnp.float32)]))
    s = _gru(tag, pidx2.reshape(ROWS, 1), pos_table, Wx[:E],
             Wx[E:].astype(jnp.bfloat16), bias.reshape(1, 3 * E),
             Wh.astype(jnp.bfloat16), bh[2 * E:].reshape(1, E),
             Wo.reshape(1, E))

    tt = jnp.arange(T)
    row_t = ((W + tt % L) * P + tt // L) * B
    self_idx = (row_t[None, :] + jnp.arange(B)[:, None])
    ante_idx = ((W + co % L) * P + co // L) * B + jnp.arange(B)[:, None]
    mix = _mix(s.reshape(ROWS), self_idx.reshape(NW, MIXN),
               ante_idx.reshape(NW, MIXN))

    return _softmax(mix)


# speedup vs baseline: 1.1447x; 1.0042x over previous
import jax
import jax.numpy as jnp
from jax import lax
from jax.experimental import pallas as pl
from jax.experimental.pallas import tpu as pltpu
from jax.experimental.pallas import tpu_sc as plsc

B = 4
T = 2048
E = 256
P = 16
L = T // P
W = 32
JT = W + L
M = P * B
ROWS = JT * M
NW = 32
GW = ROWS // NW
GISS = (128, 128, 64)
JCH = 32
NJCH = JT // JCH
MIXN = (B * T) // NW


def _sc_mesh():
    return plsc.VectorSubcoreMesh(core_axis_name="c", subcore_axis_name="s")


def _gather_body(tbl, idxh, out, idx_v, rows_v, sg, so):
    cid = lax.axis_index("c")
    sid = lax.axis_index("s")
    w = sid * 2 + cid
    base = w * GW
    offs = []
    off = 0
    for k, sz in enumerate(GISS):
        pltpu.sync_copy(idxh.at[pl.ds(base + off, sz)], idx_v[k])
        offs.append(off)
        off += sz
    gs = [pltpu.async_copy(tbl.at[idx_v[k]], rows_v[k], sg[k])
          for k in range(len(GISS))]
    os_ = []
    for k, sz in enumerate(GISS):
        gs[k].wait()
        os_.append(pltpu.async_copy(
            rows_v[k], out.at[pl.ds(base + offs[k], sz)], so[k]))
    for o in os_:
        o.wait()


def _word_gather(word_table, idx_flat):
    f = pl.kernel(
        _gather_body,
        out_type=jax.ShapeDtypeStruct((ROWS, E), jnp.float32),
        mesh=_sc_mesh(),
        scratch_types=[
            [pltpu.VMEM((sz,), jnp.int32) for sz in GISS],
            [pltpu.VMEM((sz, E), jnp.float32) for sz in GISS],
            [pltpu.SemaphoreType.DMA for _ in GISS],
            [pltpu.SemaphoreType.DMA for _ in GISS],
        ],
    )
    return f(word_table, idx_flat)


def _gru_body(tag, posq, ptab, wxp, wxw, bias, wh, bhn, wo, s_out,
              h_ref, hs_ref, pg, gx_s):
    @pl.when(pl.program_id(0) == 0)
    def _():
        h_ref[...] = jnp.zeros((M, E), jnp.float32)
        pg[...] = jnp.dot(ptab[...], wxp[...],
                          preferred_element_type=jnp.float32
                          ).astype(jnp.bfloat16)

    @pl.when(pl.program_id(0) == W // JCH)
    def _():
        h_ref[0:B, :] = jnp.zeros((B, E), jnp.float32)

    rows = tag.shape[0]
    oh = (posq[...] == lax.broadcasted_iota(jnp.int32, (rows, 64), 1)).astype(
        jnp.bfloat16)
    gx_s[...] = (jnp.dot(tag[...].astype(jnp.bfloat16), wxw[...],
                         preferred_element_type=jnp.float32)
                 + jnp.dot(oh, pg[...],
                           preferred_element_type=jnp.float32)
                 + bias[...]).astype(jnp.bfloat16)

    bhnv = bhn[...]
    wov = wo[...]

    def step(i, h):
        g = gx_s[pl.ds(i * M, M), :]
        gh = jnp.dot(h.astype(jnp.bfloat16), wh[...],
                     preferred_element_type=jnp.float32)
        r = 0.5 + 0.5 * jnp.tanh(0.5 * (g[:, 0:E] + gh[:, 0:E]))
        z = 0.5 + 0.5 * jnp.tanh(0.5 * (g[:, E:2 * E] + gh[:, E:2 * E]))
        n = jnp.tanh(g[:, 2 * E:3 * E] + r * (gh[:, 2 * E:3 * E] + bhnv))
        h2 = n + z * (h - n)
        hs_ref[pl.ds(i * M, M), :] = h2
        return h2

    h_ref[...] = lax.fori_loop(0, JCH, step, h_ref[...], unroll=8)
    s_out[...] = jnp.sum(hs_ref[...] * wov, axis=1, keepdims=True)


def _gru(tag, posq, pos_table, wxp, wxw, bias, wh, bhn, wo_row):
    rows = JCH * M
    return pl.pallas_call(
        _gru_body,
        grid=(NJCH,),
        in_specs=[
            pl.BlockSpec((rows, E), lambda i: (i, 0)),
            pl.BlockSpec((rows, 1), lambda i: (i, 0)),
            pl.BlockSpec((64, E), lambda i: (0, 0)),
            pl.BlockSpec((E, 3 * E), lambda i: (0, 0)),
            pl.BlockSpec((E, 3 * E), lambda i: (0, 0)),
            pl.BlockSpec((1, 3 * E), lambda i: (0, 0)),
            pl.BlockSpec((E, 3 * E), lambda i: (0, 0)),
            pl.BlockSpec((1, E), lambda i: (0, 0)),
            pl.BlockSpec((1, E), lambda i: (0, 0)),
        ],
        out_specs=pl.BlockSpec((rows, 1), lambda i: (i, 0)),
        out_shape=jax.ShapeDtypeStruct((ROWS, 1), jnp.float32),
        scratch_shapes=[
            pltpu.VMEM((M, E), jnp.float32),
            pltpu.VMEM((rows, E), jnp.float32),
            pltpu.VMEM((64, 3 * E), jnp.bfloat16),
            pltpu.VMEM((rows, 3 * E), jnp.bfloat16),
        ],
    )(tag, posq, pos_table, wxp, wxw, bias, wh, bhn, wo_row)


def _mix_body(s_h, sidx_h, aidx_h, out_h, s_v, sidx_v, aidx_v, out_v):
    cid = lax.axis_index("c")
    sid = lax.axis_index("s")
    w = sid * 2 + cid
    pltpu.sync_copy(s_h, s_v)
    pltpu.sync_copy(sidx_h.at[w], sidx_v)
    pltpu.sync_copy(aidx_h.at[w], aidx_v)
    for j in range(MIXN // 16):
        si = sidx_v[pl.ds(j * 16, 16)]
        ai = aidx_v[pl.ds(j * 16, 16)]
        gs = plsc.load_gather(s_v, [si])
        ga = plsc.load_gather(s_v, [ai])
        out_v[pl.ds(j * 16, 16)] = 0.5 * (gs + ga)
    b = w // (T // MIXN)
    tcol = (w % (T // MIXN)) * MIXN
    pltpu.sync_copy(out_v, out_h.at[b, pl.ds(tcol, MIXN)])


def _mix(s_flat, self_rs, ante_rs):
    f = pl.kernel(
        _mix_body,
        out_type=jax.ShapeDtypeStruct((B, T), jnp.float32),
        mesh=_sc_mesh(),
        compiler_params=pltpu.CompilerParams(needs_layout_passes=False),
        scratch_types=[
            pltpu.VMEM((ROWS,), jnp.float32),
            pltpu.VMEM((MIXN,), jnp.int32),
            pltpu.VMEM((MIXN,), jnp.int32),
            pltpu.VMEM((MIXN,), jnp.float32),
        ],
    )
    return f(s_flat, self_rs, ante_rs)


def _sm_body(m, o):
    v = m[...]
    mx = jnp.max(v, axis=1, keepdims=True)
    e = jnp.exp(v - mx)
    o[...] = e / jnp.sum(e, axis=1, keepdims=True)


def _softmax(mix2):
    return pl.pallas_call(
        _sm_body,
        out_shape=jax.ShapeDtypeStruct((B, T), jnp.float32),
    )(mix2)


def kernel(x, co, pos_table, word_table, Wx, Wh, bx, bh, Wo, bo):
    pos_idx = x[:, :, 0]
    word_idx = x[:, :, 1]

    def seg_layout(idx_bt):
        it = idx_bt.T
        warm = jnp.roll(it, W, axis=0).reshape(P, L, B)[:, :W]
        main = it.reshape(P, L, B)
        return jnp.concatenate([warm, main], axis=1).transpose(1, 0, 2)

    widx2 = seg_layout(word_idx)
    pidx2 = seg_layout(pos_idx)

    tag = _word_gather(word_table, widx2.reshape(ROWS))

    bias = (bx + jnp.concatenate([bh[:2 * E], jnp.zeros((E,), jnp.float32)]))
    s = _gru(tag, pidx2.reshape(ROWS, 1), pos_table, Wx[:E],
             Wx[E:].astype(jnp.bfloat16), bias.reshape(1, 3 * E),
             Wh.astype(jnp.bfloat16), bh[2 * E:].reshape(1, E),
             Wo.reshape(1, E))

    tt = jnp.arange(T)
    row_t = ((W + tt % L) * P + tt // L) * B
    self_idx = (row_t[None, :] + jnp.arange(B)[:, None])
    ante_idx = ((W + co % L) * P + co // L) * B + jnp.arange(B)[:, None]
    mix = _mix(s.reshape(ROWS), self_idx.reshape(NW, MIXN),
               ante_idx.reshape(NW, MIXN))

    return _softmax(mix)
